# Initial kernel scaffold; baseline (speedup 1.0000x reference)
import jax
import jax.numpy as jnp
from jax.experimental import pallas as pl


def kernel(q):
    raise NotImplementedError("write your pallas kernel here")


# speedup vs baseline: 50.4776x; 50.4776x over previous
import functools

import jax
import jax.numpy as jnp
from jax import lax
from jax.experimental import pallas as pl

N = 4096
D = 128
BLK = 256
K = 16
EPS = 1.0
BIG = 3.0e38


def _row_losses(d2):
    c = jnp.maximum(EPS - jnp.sqrt(d2), 0.0)
    s_all = jnp.sum(c, axis=1, keepdims=True)
    m0 = jnp.min(d2, axis=1, keepdims=True)
    drop = jnp.maximum(EPS - jnp.sqrt(m0), 0.0)
    return s_all - drop


def _block_kernel(a_ref, qf_ref, out_ref):
    i = pl.program_id(0)

    a = a_ref[...]
    qf = qf_ref[...]
    a2 = jnp.sum(a * a, axis=1, keepdims=True)
    b2 = jnp.sum(qf * qf, axis=1, keepdims=True)
    ab = lax.dot_general(a, qf, (((1,), (1,)), ((), ())),
                         preferred_element_type=jnp.float32)
    d2 = jnp.maximum(a2 + jnp.transpose(b2) - 2.0 * ab, 0.0)

    m0 = jnp.min(d2, axis=1, keepdims=True)
    thr = jnp.float32(EPS * EPS)
    cnt = jnp.sum((d2 < thr).astype(jnp.float32), axis=1, keepdims=True)
    cnt_excl = cnt - (m0 < thr).astype(jnp.float32)

    fast = _row_losses(d2)

    @pl.when(i == 0)
    def _init():
        out_ref[...] = jnp.zeros((1, 1), jnp.float32)

    partial = jnp.sum(fast, keepdims=True).reshape(1, 1) * jnp.float32(1.0 / N)

    need_slow = jnp.max(cnt_excl) > jnp.float32(K)

    @pl.when(jnp.logical_not(need_slow))
    def _fast():
        out_ref[...] += partial

    @pl.when(need_slow)
    def _slow():
        def body(_, carry):
            vals, rem, acc = carry
            m = jnp.min(vals, axis=1, keepdims=True)
            ismin = vals == m
            cmult = jnp.sum(ismin.astype(jnp.float32), axis=1, keepdims=True)
            take = jnp.minimum(cmult, rem)
            acc = acc + take * jnp.maximum(EPS - jnp.sqrt(m), 0.0)
            rem = rem - take
            vals = jnp.where(ismin, BIG, vals)
            return vals, rem, acc

        rem0 = jnp.full((BLK, 1), jnp.float32(K + 1))
        acc0 = jnp.zeros((BLK, 1), jnp.float32)
        _, _, acc = lax.fori_loop(0, K + 1, body, (d2, rem0, acc0))
        acc = acc - jnp.maximum(EPS - jnp.sqrt(m0), 0.0)
        out_ref[...] += jnp.sum(acc, keepdims=True).reshape(1, 1) * jnp.float32(1.0 / N)


@jax.jit
def kernel(q):
    out = pl.pallas_call(
        _block_kernel,
        grid=(N // BLK,),
        in_specs=[
            pl.BlockSpec((BLK, D), lambda i: (i, 0)),
            pl.BlockSpec((N, D), lambda i: (0, 0)),
        ],
        out_specs=pl.BlockSpec((1, 1), lambda i: (0, 0)),
        out_shape=jax.ShapeDtypeStruct((1, 1), jnp.float32),
    )(q, q)
    return out[0, 0]


# speedup vs baseline: 58.2255x; 1.1535x over previous
import jax
import jax.numpy as jnp
from jax import lax
from jax.experimental import pallas as pl
from jax.experimental.pallas import tpu as pltpu

N = 4096
D = 128
BLK = 256
K = 16
EPS = 1.0
BIG = 3.0e38


def _block_kernel(a_ref, qf_ref, out_ref, b2r_ref, b2c_ref):
    i = pl.program_id(0)

    @pl.when(i == 0)
    def _init():
        qf = qf_ref[...]
        b2c = jnp.sum(qf * qf, axis=1, keepdims=True)
        b2c_ref[...] = b2c
        b2r_ref[...] = jnp.transpose(b2c)
        out_ref[...] = jnp.zeros((1, 1), jnp.float32)

    a = a_ref[...]
    ab = lax.dot_general(a, qf_ref[...], (((1,), (1,)), ((), ())),
                         preferred_element_type=jnp.float32)
    a2 = b2c_ref[pl.ds(i * BLK, BLK), :]
    d2 = a2 + b2r_ref[...] - 2.0 * ab

    thr = jnp.float32(EPS * EPS)

    @pl.when(jnp.min(d2) < thr)
    def _full():
        d2c = jnp.maximum(d2, 0.0)
        m0 = jnp.min(d2c, axis=1, keepdims=True)
        drop = jnp.maximum(EPS - jnp.sqrt(m0), 0.0)
        c = jnp.maximum(EPS - jnp.sqrt(d2c), 0.0)
        s_all = jnp.sum(c, axis=1, keepdims=True)
        cnt = jnp.sum((d2c < thr).astype(jnp.float32), axis=1, keepdims=True)
        cnt_excl = cnt - (m0 < thr).astype(jnp.float32)
        fast = s_all - drop
        need_slow = jnp.max(cnt_excl) > jnp.float32(K)

        @pl.when(jnp.logical_not(need_slow))
        def _fast():
            out_ref[...] += jnp.sum(fast, keepdims=True).reshape(1, 1) \
                * jnp.float32(1.0 / N)

        @pl.when(need_slow)
        def _slow():
            def body(_, carry):
                vals, rem, acc = carry
                m = jnp.min(vals, axis=1, keepdims=True)
                ismin = vals == m
                cmult = jnp.sum(ismin.astype(jnp.float32), axis=1,
                                keepdims=True)
                take = jnp.minimum(cmult, rem)
                acc = acc + take * jnp.maximum(EPS - jnp.sqrt(m), 0.0)
                rem = rem - take
                vals = jnp.where(ismin, BIG, vals)
                return vals, rem, acc

            rem0 = jnp.full((BLK, 1), jnp.float32(K + 1))
            acc0 = jnp.zeros((BLK, 1), jnp.float32)
            _, _, acc = lax.fori_loop(0, K + 1, body, (d2c, rem0, acc0))
            acc = acc - drop
            out_ref[...] += jnp.sum(acc, keepdims=True).reshape(1, 1) \
                * jnp.float32(1.0 / N)


@jax.jit
def kernel(q):
    out = pl.pallas_call(
        _block_kernel,
        grid=(N // BLK,),
        in_specs=[
            pl.BlockSpec((BLK, D), lambda i: (i, 0)),
            pl.BlockSpec((N, D), lambda i: (0, 0)),
        ],
        out_specs=pl.BlockSpec((1, 1), lambda i: (0, 0)),
        out_shape=jax.ShapeDtypeStruct((1, 1), jnp.float32),
        scratch_shapes=[
            pltpu.VMEM((1, N), jnp.float32),
            pltpu.VMEM((N, 1), jnp.float32),
        ],
    )(q, q)
    return out[0, 0]


# speedup vs baseline: 62.6783x; 1.0765x over previous
import jax
import jax.numpy as jnp
from jax import lax
from jax.experimental import pallas as pl
from jax.experimental.pallas import tpu as pltpu

N = 4096
D = 128
BLK = 256
K = 16
EPS = 1.0
BIG = 3.0e38


def _block_kernel(a_ref, qf_ref, out_ref, b2r_ref, b2c_ref):
    i = pl.program_id(0)

    @pl.when(i == 0)
    def _init():
        qf = qf_ref[...]
        b2c = jnp.sum(qf * qf, axis=1, keepdims=True)
        b2c_ref[...] = b2c
        b2r_ref[...] = jnp.transpose(b2c)
        out_ref[...] = jnp.zeros((1, 1), jnp.float32)

    a = a_ref[...] * jnp.float32(-2.0)
    ab = lax.dot_general(a, qf_ref[...], (((1,), (1,)), ((), ())),
                         preferred_element_type=jnp.float32)
    a2 = b2c_ref[pl.ds(i * BLK, BLK), :]
    e = ab + b2r_ref[...]
    m_rows = jnp.min(e, axis=1, keepdims=True) + a2

    thr = jnp.float32(EPS * EPS)

    @pl.when(jnp.min(m_rows) < thr)
    def _full():
        d2c = jnp.maximum(e + a2, 0.0)
        m0 = jnp.min(d2c, axis=1, keepdims=True)
        drop = jnp.maximum(EPS - jnp.sqrt(m0), 0.0)
        c = jnp.maximum(EPS - jnp.sqrt(d2c), 0.0)
        s_all = jnp.sum(c, axis=1, keepdims=True)
        cnt = jnp.sum((d2c < thr).astype(jnp.float32), axis=1, keepdims=True)
        cnt_excl = cnt - (m0 < thr).astype(jnp.float32)
        fast = s_all - drop
        need_slow = jnp.max(cnt_excl) > jnp.float32(K)

        @pl.when(jnp.logical_not(need_slow))
        def _fast():
            out_ref[...] += jnp.sum(fast, keepdims=True).reshape(1, 1) \
                * jnp.float32(1.0 / N)

        @pl.when(need_slow)
        def _slow():
            def body(_, carry):
                vals, rem, acc = carry
                m = jnp.min(vals, axis=1, keepdims=True)
                ismin = vals == m
                cmult = jnp.sum(ismin.astype(jnp.float32), axis=1,
                                keepdims=True)
                take = jnp.minimum(cmult, rem)
                acc = acc + take * jnp.maximum(EPS - jnp.sqrt(m), 0.0)
                rem = rem - take
                vals = jnp.where(ismin, BIG, vals)
                return vals, rem, acc

            rem0 = jnp.full((BLK, 1), jnp.float32(K + 1))
            acc0 = jnp.zeros((BLK, 1), jnp.float32)
            _, _, acc = lax.fori_loop(0, K + 1, body, (d2c, rem0, acc0))
            acc = acc - drop
            out_ref[...] += jnp.sum(acc, keepdims=True).reshape(1, 1) \
                * jnp.float32(1.0 / N)


@jax.jit
def kernel(q):
    out = pl.pallas_call(
        _block_kernel,
        grid=(N // BLK,),
        in_specs=[
            pl.BlockSpec((BLK, D), lambda i: (i, 0)),
            pl.BlockSpec((N, D), lambda i: (0, 0)),
        ],
        out_specs=pl.BlockSpec((1, 1), lambda i: (0, 0)),
        out_shape=jax.ShapeDtypeStruct((1, 1), jnp.float32),
        scratch_shapes=[
            pltpu.VMEM((1, N), jnp.float32),
            pltpu.VMEM((N, 1), jnp.float32),
        ],
    )(q, q)
    return out[0, 0]


# speedup vs baseline: 68.8048x; 1.0977x over previous
import jax
import jax.numpy as jnp
from jax import lax
from jax.experimental import pallas as pl
from jax.experimental.pallas import tpu as pltpu

N = 4096
D = 128
BLK = 512
K = 16
EPS = 1.0
BIG = 3.0e38


def _block_kernel(a_ref, qf_ref, out_ref, b2r_ref, b2c_ref):
    i = pl.program_id(0)

    @pl.when(i == 0)
    def _init():
        qf = qf_ref[...]
        b2c = jnp.sum(qf * qf, axis=1, keepdims=True)
        b2c_ref[...] = b2c
        b2r_ref[...] = jnp.transpose(b2c)
        out_ref[...] = jnp.zeros((1, 1), jnp.float32)

    a = a_ref[...] * jnp.float32(-2.0)
    ab = lax.dot_general(a, qf_ref[...], (((1,), (1,)), ((), ())),
                         preferred_element_type=jnp.float32)
    a2 = b2c_ref[pl.ds(i * BLK, BLK), :]
    e = ab + b2r_ref[...]
    m_rows = jnp.min(e, axis=1, keepdims=True) + a2

    thr = jnp.float32(EPS * EPS)

    @pl.when(jnp.min(m_rows) < thr)
    def _full():
        d2c = jnp.maximum(e + a2, 0.0)
        m0 = jnp.min(d2c, axis=1, keepdims=True)
        drop = jnp.maximum(EPS - jnp.sqrt(m0), 0.0)
        c = jnp.maximum(EPS - jnp.sqrt(d2c), 0.0)
        s_all = jnp.sum(c, axis=1, keepdims=True)
        cnt = jnp.sum((d2c < thr).astype(jnp.float32), axis=1, keepdims=True)
        cnt_excl = cnt - (m0 < thr).astype(jnp.float32)
        fast = s_all - drop
        need_slow = jnp.max(cnt_excl) > jnp.float32(K)

        @pl.when(jnp.logical_not(need_slow))
        def _fast():
            out_ref[...] += jnp.sum(fast, keepdims=True).reshape(1, 1) \
                * jnp.float32(1.0 / N)

        @pl.when(need_slow)
        def _slow():
            def body(_, carry):
                vals, rem, acc = carry
                m = jnp.min(vals, axis=1, keepdims=True)
                ismin = vals == m
                cmult = jnp.sum(ismin.astype(jnp.float32), axis=1,
                                keepdims=True)
                take = jnp.minimum(cmult, rem)
                acc = acc + take * jnp.maximum(EPS - jnp.sqrt(m), 0.0)
                rem = rem - take
                vals = jnp.where(ismin, BIG, vals)
                return vals, rem, acc

            rem0 = jnp.full((BLK, 1), jnp.float32(K + 1))
            acc0 = jnp.zeros((BLK, 1), jnp.float32)
            _, _, acc = lax.fori_loop(0, K + 1, body, (d2c, rem0, acc0))
            acc = acc - drop
            out_ref[...] += jnp.sum(acc, keepdims=True).reshape(1, 1) \
                * jnp.float32(1.0 / N)


@jax.jit
def kernel(q):
    out = pl.pallas_call(
        _block_kernel,
        grid=(N // BLK,),
        in_specs=[
            pl.BlockSpec((BLK, D), lambda i: (i, 0)),
            pl.BlockSpec((N, D), lambda i: (0, 0)),
        ],
        out_specs=pl.BlockSpec((1, 1), lambda i: (0, 0)),
        out_shape=jax.ShapeDtypeStruct((1, 1), jnp.float32),
        scratch_shapes=[
            pltpu.VMEM((1, N), jnp.float32),
            pltpu.VMEM((N, 1), jnp.float32),
        ],
    )(q, q)
    return out[0, 0]


# speedup vs baseline: 68.8925x; 1.0013x over previous
import jax
import jax.numpy as jnp
from jax import lax
from jax.experimental import pallas as pl
from jax.experimental.pallas import tpu as pltpu

N = 4096
D = 128
BLK = 512
K = 16
EPS = 1.0
BIG = 3.0e38


def _block_kernel(a_ref, qf_ref, out_ref, b2r_ref, b2c_ref):
    i = pl.program_id(0)

    @pl.when(i == 0)
    def _init():
        qf = qf_ref[...]
        b2c = jnp.sum(qf * qf, axis=1, keepdims=True)
        b2c_ref[...] = b2c
        b2r_ref[...] = jnp.transpose(b2c)
        out_ref[...] = jnp.zeros((1, 1), jnp.float32)

    a = a_ref[...] * jnp.float32(-2.0)
    ab = lax.dot_general(a, qf_ref[...], (((1,), (1,)), ((), ())),
                         preferred_element_type=jnp.float32)
    a2 = b2c_ref[pl.ds(i * BLK, BLK), :]
    b2r = b2r_ref[...]
    m_rows = jnp.min(ab + b2r, axis=1, keepdims=True) + a2

    thr = jnp.float32(EPS * EPS)

    @pl.when(jnp.min(m_rows) < thr)
    def _full():
        d2c = jnp.maximum((ab + b2r) + a2, 0.0)
        m0 = jnp.min(d2c, axis=1, keepdims=True)
        drop = jnp.maximum(EPS - jnp.sqrt(m0), 0.0)
        c = jnp.maximum(EPS - jnp.sqrt(d2c), 0.0)
        s_all = jnp.sum(c, axis=1, keepdims=True)
        cnt = jnp.sum((d2c < thr).astype(jnp.float32), axis=1, keepdims=True)
        cnt_excl = cnt - (m0 < thr).astype(jnp.float32)
        fast = s_all - drop
        need_slow = jnp.max(cnt_excl) > jnp.float32(K)

        @pl.when(jnp.logical_not(need_slow))
        def _fast():
            out_ref[...] += jnp.sum(fast, keepdims=True).reshape(1, 1) \
                * jnp.float32(1.0 / N)

        @pl.when(need_slow)
        def _slow():
            def body(_, carry):
                vals, rem, acc = carry
                m = jnp.min(vals, axis=1, keepdims=True)
                ismin = vals == m
                cmult = jnp.sum(ismin.astype(jnp.float32), axis=1,
                                keepdims=True)
                take = jnp.minimum(cmult, rem)
                acc = acc + take * jnp.maximum(EPS - jnp.sqrt(m), 0.0)
                rem = rem - take
                vals = jnp.where(ismin, BIG, vals)
                return vals, rem, acc

            rem0 = jnp.full((BLK, 1), jnp.float32(K + 1))
            acc0 = jnp.zeros((BLK, 1), jnp.float32)
            _, _, acc = lax.fori_loop(0, K + 1, body, (d2c, rem0, acc0))
            acc = acc - drop
            out_ref[...] += jnp.sum(acc, keepdims=True).reshape(1, 1) \
                * jnp.float32(1.0 / N)


@jax.jit
def kernel(q):
    out = pl.pallas_call(
        _block_kernel,
        grid=(N // BLK,),
        in_specs=[
            pl.BlockSpec((BLK, D), lambda i: (i, 0)),
            pl.BlockSpec((N, D), lambda i: (0, 0)),
        ],
        out_specs=pl.BlockSpec((1, 1), lambda i: (0, 0)),
        out_shape=jax.ShapeDtypeStruct((1, 1), jnp.float32),
        scratch_shapes=[
            pltpu.VMEM((1, N), jnp.float32),
            pltpu.VMEM((N, 1), jnp.float32),
        ],
    )(q, q)
    return out[0, 0]


# speedup vs baseline: 72.2995x; 1.0495x over previous
import jax
import jax.numpy as jnp
from jax import lax
from jax.experimental import pallas as pl
from jax.experimental.pallas import tpu as pltpu

N = 4096
D = 128
BLK = 1024
K = 16
EPS = 1.0
BIG = 3.0e38


def _block_kernel(a_ref, qf_ref, out_ref, b2r_ref, b2c_ref):
    i = pl.program_id(0)

    @pl.when(i == 0)
    def _init():
        qf = qf_ref[...]
        b2c = jnp.sum(qf * qf, axis=1, keepdims=True)
        b2c_ref[...] = b2c
        b2r_ref[...] = jnp.transpose(b2c)
        out_ref[...] = jnp.zeros((1, 1), jnp.float32)

    a = a_ref[...] * jnp.float32(-2.0)
    ab = lax.dot_general(a, qf_ref[...], (((1,), (1,)), ((), ())),
                         preferred_element_type=jnp.float32)
    a2 = b2c_ref[pl.ds(i * BLK, BLK), :]
    b2r = b2r_ref[...]
    m_rows = jnp.min(ab + b2r, axis=1, keepdims=True) + a2

    thr = jnp.float32(EPS * EPS)

    @pl.when(jnp.min(m_rows) < thr)
    def _full():
        d2c = jnp.maximum((ab + b2r) + a2, 0.0)
        m0 = jnp.min(d2c, axis=1, keepdims=True)
        drop = jnp.maximum(EPS - jnp.sqrt(m0), 0.0)
        c = jnp.maximum(EPS - jnp.sqrt(d2c), 0.0)
        s_all = jnp.sum(c, axis=1, keepdims=True)
        cnt = jnp.sum((d2c < thr).astype(jnp.float32), axis=1, keepdims=True)
        cnt_excl = cnt - (m0 < thr).astype(jnp.float32)
        fast = s_all - drop
        need_slow = jnp.max(cnt_excl) > jnp.float32(K)

        @pl.when(jnp.logical_not(need_slow))
        def _fast():
            out_ref[...] += jnp.sum(fast, keepdims=True).reshape(1, 1) \
                * jnp.float32(1.0 / N)

        @pl.when(need_slow)
        def _slow():
            def body(_, carry):
                vals, rem, acc = carry
                m = jnp.min(vals, axis=1, keepdims=True)
                ismin = vals == m
                cmult = jnp.sum(ismin.astype(jnp.float32), axis=1,
                                keepdims=True)
                take = jnp.minimum(cmult, rem)
                acc = acc + take * jnp.maximum(EPS - jnp.sqrt(m), 0.0)
                rem = rem - take
                vals = jnp.where(ismin, BIG, vals)
                return vals, rem, acc

            rem0 = jnp.full((BLK, 1), jnp.float32(K + 1))
            acc0 = jnp.zeros((BLK, 1), jnp.float32)
            _, _, acc = lax.fori_loop(0, K + 1, body, (d2c, rem0, acc0))
            acc = acc - drop
            out_ref[...] += jnp.sum(acc, keepdims=True).reshape(1, 1) \
                * jnp.float32(1.0 / N)


@jax.jit
def kernel(q):
    out = pl.pallas_call(
        _block_kernel,
        grid=(N // BLK,),
        in_specs=[
            pl.BlockSpec((BLK, D), lambda i: (i, 0)),
            pl.BlockSpec((N, D), lambda i: (0, 0)),
        ],
        out_specs=pl.BlockSpec((1, 1), lambda i: (0, 0)),
        out_shape=jax.ShapeDtypeStruct((1, 1), jnp.float32),
        scratch_shapes=[
            pltpu.VMEM((1, N), jnp.float32),
            pltpu.VMEM((N, 1), jnp.float32),
        ],
    )(q, q)
    return out[0, 0]
